# Initial kernel scaffold; baseline (speedup 1.0000x reference)
import jax
import jax.numpy as jnp
from jax.experimental import pallas as pl


def kernel(logits):
    raise NotImplementedError("write your pallas kernel here")


# speedup vs baseline: 14.7512x; 14.7512x over previous
import functools

import jax
import jax.numpy as jnp
from jax import lax
from jax.experimental import pallas as pl
from jax.experimental.pallas import tpu as pltpu
from jax.experimental.pallas import tpu_sc as plsc

K_TOP = 2048
TEMPERATURE = 0.1

N = 4194304
NUM_CORES = 2
NUM_SUBCORES = 16
NUM_WORKERS = NUM_CORES * NUM_SUBCORES
LANES = 16
SHARD = N // NUM_WORKERS
CHUNK = 16384
NCHUNK = SHARD // CHUNK
BINS = 4096
MIN_I32 = -(2**31)


def _monotone_key(x_f32):
    b = lax.bitcast_convert_type(x_f32, jnp.int32)
    return b ^ ((b >> 31) | jnp.int32(MIN_I32))


def _make_hist_kernel(shift, prefix_shift):
    mesh = plsc.VectorSubcoreMesh(core_axis_name="c", subcore_axis_name="s")

    @functools.partial(
        pl.kernel,
        mesh=mesh,
        out_type=jax.ShapeDtypeStruct((NUM_WORKERS, BINS), jnp.int32),
        compiler_params=pltpu.CompilerParams(needs_layout_passes=False),
        scratch_types=[
            pltpu.VMEM((LANES * BINS,), jnp.int32),
            pltpu.VMEM((CHUNK,), jnp.float32),
            pltpu.VMEM((LANES,), jnp.int32),
            pltpu.VMEM((BINS,), jnp.int32),
        ],
    )
    def hist_kernel(logits_hbm, prefix_hbm, out_hbm, hist_v, buf_v, pref_v, red_v):
        wid = lax.axis_index("s") * NUM_CORES + lax.axis_index("c")
        base = wid * SHARD
        lane = lax.iota(jnp.int32, LANES)
        ones = jnp.ones((LANES,), jnp.int32)
        zeros = jnp.zeros((LANES,), jnp.int32)

        pltpu.sync_copy(prefix_hbm, pref_v)
        pvec = pref_v[...]

        def zero_body(j, _):
            hist_v[pl.ds(j * LANES, LANES)] = zeros
            return _

        lax.fori_loop(0, LANES * BINS // LANES, zero_body, None)

        def elem_body(i, _):
            x = buf_v[pl.ds(i * LANES, LANES)]
            key = _monotone_key(x)
            b = lax.shift_right_logical(key, shift) & (BINS - 1)
            addr = lane * BINS + b
            if prefix_shift is None:
                plsc.addupdate_scatter(hist_v, [addr], ones)
            else:
                match = lax.shift_right_logical(key, prefix_shift) == pvec
                plsc.addupdate_scatter(hist_v, [addr], ones, mask=match)
            return _

        for c in range(NCHUNK):
            pltpu.sync_copy(logits_hbm.at[pl.ds(base + c * CHUNK, CHUNK)], buf_v)
            lax.fori_loop(0, CHUNK // LANES, elem_body, None)

        def red_body(j, _):
            acc = hist_v[pl.ds(j * LANES, LANES)]
            for l in range(1, LANES):
                acc = acc + hist_v[pl.ds(l * BINS + j * LANES, LANES)]
            red_v[pl.ds(j * LANES, LANES)] = acc
            return _

        lax.fori_loop(0, BINS // LANES, red_body, None)
        pltpu.sync_copy(red_v, out_hbm.at[wid])

    return hist_kernel


_hist_pass1 = _make_hist_kernel(shift=20, prefix_shift=None)
_hist_pass2 = _make_hist_kernel(shift=8, prefix_shift=20)
_hist_pass3 = _make_hist_kernel(shift=0, prefix_shift=8)


def _pick_bucket(hist_workers, rank):
    h = jnp.sum(hist_workers, axis=0)
    cum = jnp.cumsum(h)
    b = jnp.argmax(cum >= rank + 1).astype(jnp.int32)
    new_rank = rank - (cum[b] - h[b])
    return b, new_rank


def _sigmoid_body(x_ref, t_ref, o_ref):
    t = t_ref[0]
    z = (t - x_ref[...]) * jnp.float32(1.0 / TEMPERATURE)
    o_ref[...] = 1.0 / (1.0 + jnp.exp(z))


def kernel(logits):
    n = logits.shape[-1]
    rank = jnp.int32(n - K_TOP - 1)

    zeros16 = jnp.zeros((LANES,), jnp.int32)
    h1 = _hist_pass1(logits, zeros16)
    b1, rank = _pick_bucket(h1, rank)

    pref1 = jnp.full((LANES,), b1, jnp.int32)
    h2 = _hist_pass2(logits, pref1)
    b2, rank = _pick_bucket(h2, rank)

    pref2 = jnp.full((LANES,), (b1 << 12) | b2, jnp.int32)
    h3 = _hist_pass3(logits, pref2)
    b3, rank = _pick_bucket(h3, rank)

    key = (b1 << 20) | (b2 << 8) | b3
    vbits = jnp.where(key < 0, key ^ jnp.int32(MIN_I32), ~key)
    kth_value = lax.bitcast_convert_type(vbits, jnp.float32)

    rows = 4096
    cols = n // rows
    block_rows = 512
    x2 = logits.reshape(rows, cols)
    t1 = kth_value.reshape(1)
    out = pl.pallas_call(
        _sigmoid_body,
        grid=(rows // block_rows,),
        in_specs=[
            pl.BlockSpec((block_rows, cols), lambda i: (i, 0)),
            pl.BlockSpec(memory_space=pltpu.SMEM),
        ],
        out_specs=pl.BlockSpec((block_rows, cols), lambda i: (i, 0)),
        out_shape=jax.ShapeDtypeStruct((rows, cols), jnp.float32),
    )(x2, t1)
    return out.reshape(n)


# speedup vs baseline: 25.0812x; 1.7003x over previous
import functools

import jax
import jax.numpy as jnp
from jax import lax
from jax.experimental import pallas as pl
from jax.experimental.pallas import tpu as pltpu
from jax.experimental.pallas import tpu_sc as plsc

K_TOP = 2048
TEMPERATURE = 0.1

N = 4194304
NUM_CORES = 2
NUM_SUBCORES = 16
NUM_WORKERS = NUM_CORES * NUM_SUBCORES
LANES = 16
SHARD = N // NUM_WORKERS
CHUNK = 16384
NCHUNK = SHARD // CHUNK
BINS = 65536
UNROLL = 8
MIN_I32 = -(2**31)


def _monotone_key(x_f32):
    b = lax.bitcast_convert_type(x_f32, jnp.int32)
    return b ^ ((b >> 31) | jnp.int32(MIN_I32))


def _make_hist_kernel(shift, prefix_shift):
    mesh = plsc.VectorSubcoreMesh(core_axis_name="c", subcore_axis_name="s")

    @functools.partial(
        pl.kernel,
        mesh=mesh,
        out_type=jax.ShapeDtypeStruct((NUM_WORKERS, BINS), jnp.int32),
        compiler_params=pltpu.CompilerParams(needs_layout_passes=False),
        scratch_types=[
            pltpu.VMEM((BINS,), jnp.int32),
            pltpu.VMEM((CHUNK,), jnp.float32),
            pltpu.VMEM((LANES,), jnp.int32),
        ],
    )
    def hist_kernel(logits_hbm, prefix_hbm, out_hbm, hist_v, buf_v, pref_v):
        wid = lax.axis_index("s") * NUM_CORES + lax.axis_index("c")
        base = wid * SHARD
        ones = jnp.ones((LANES,), jnp.int32)
        zeros = jnp.zeros((LANES,), jnp.int32)

        pltpu.sync_copy(prefix_hbm, pref_v)
        pvec = pref_v[...]

        def zero_body(j, _):
            for u in range(UNROLL):
                hist_v[pl.ds((j * UNROLL + u) * LANES, LANES)] = zeros
            return _

        lax.fori_loop(0, BINS // LANES // UNROLL, zero_body, None)

        def elem_body(i, _):
            for u in range(UNROLL):
                x = buf_v[pl.ds((i * UNROLL + u) * LANES, LANES)]
                key = _monotone_key(x)
                b = lax.shift_right_logical(key, shift) & (BINS - 1)
                if prefix_shift is None:
                    plsc.addupdate_scatter(hist_v, [b], ones)
                else:
                    match = lax.shift_right_logical(key, prefix_shift) == pvec
                    plsc.addupdate_scatter(hist_v, [b], ones, mask=match)
            return _

        for c in range(NCHUNK):
            pltpu.sync_copy(logits_hbm.at[pl.ds(base + c * CHUNK, CHUNK)], buf_v)
            lax.fori_loop(0, CHUNK // LANES // UNROLL, elem_body, None)

        pltpu.sync_copy(hist_v, out_hbm.at[wid])

    return hist_kernel


_hist_pass1 = _make_hist_kernel(shift=16, prefix_shift=None)
_hist_pass2 = _make_hist_kernel(shift=0, prefix_shift=16)


def _pick_bucket(hist_workers, rank):
    h = jnp.sum(hist_workers, axis=0)
    cum = jnp.cumsum(h)
    b = jnp.argmax(cum >= rank + 1).astype(jnp.int32)
    new_rank = rank - (cum[b] - h[b])
    return b, new_rank


def _sigmoid_body(x_ref, t_ref, o_ref):
    t = t_ref[0]
    z = (t - x_ref[...]) * jnp.float32(1.0 / TEMPERATURE)
    o_ref[...] = 1.0 / (1.0 + jnp.exp(z))


def kernel(logits):
    n = logits.shape[-1]
    rank = jnp.int32(n - K_TOP - 1)

    zeros16 = jnp.zeros((LANES,), jnp.int32)
    h1 = _hist_pass1(logits, zeros16)
    b1, rank = _pick_bucket(h1, rank)

    pref1 = jnp.full((LANES,), b1, jnp.int32)
    h2 = _hist_pass2(logits, pref1)
    b2, rank = _pick_bucket(h2, rank)

    key = (b1 << 16) | b2
    vbits = jnp.where(key < 0, key ^ jnp.int32(MIN_I32), ~key)
    kth_value = lax.bitcast_convert_type(vbits, jnp.float32)

    rows = 4096
    cols = n // rows
    block_rows = 512
    x2 = logits.reshape(rows, cols)
    t1 = kth_value.reshape(1)
    out = pl.pallas_call(
        _sigmoid_body,
        grid=(rows // block_rows,),
        in_specs=[
            pl.BlockSpec((block_rows, cols), lambda i: (i, 0)),
            pl.BlockSpec(memory_space=pltpu.SMEM),
        ],
        out_specs=pl.BlockSpec((block_rows, cols), lambda i: (i, 0)),
        out_shape=jax.ShapeDtypeStruct((rows, cols), jnp.float32),
    )(x2, t1)
    return out.reshape(n)


# speedup vs baseline: 56.2252x; 2.2417x over previous
import functools

import jax
import jax.numpy as jnp
from jax import lax
from jax.experimental import pallas as pl
from jax.experimental.pallas import tpu as pltpu
from jax.experimental.pallas import tpu_sc as plsc

K_TOP = 2048
TEMPERATURE = 0.1

N = 4194304
NUM_CORES = 2
NUM_SUBCORES = 16
NUM_WORKERS = NUM_CORES * NUM_SUBCORES
LANES = 16
SHARD = N // NUM_WORKERS
CHUNK = 16384
NCHUNK = SHARD // CHUNK
BINS = 65536
UNROLL = 8
MIN_I32 = -(2**31)


def _make_hist_kernel(prefix_pass):
    mesh = plsc.VectorSubcoreMesh(core_axis_name="c", subcore_axis_name="s")

    @functools.partial(
        pl.kernel,
        mesh=mesh,
        out_type=jax.ShapeDtypeStruct((NUM_WORKERS, BINS), jnp.int32),
        compiler_params=pltpu.CompilerParams(needs_layout_passes=False),
        scratch_types=[
            pltpu.VMEM((BINS,), jnp.int32),
            pltpu.VMEM((CHUNK,), jnp.float32),
            pltpu.VMEM((CHUNK,), jnp.float32),
            pltpu.VMEM((LANES,), jnp.int32),
            pltpu.SemaphoreType.DMA,
            pltpu.SemaphoreType.DMA,
        ],
    )
    def hist_kernel(logits_hbm, prefix_hbm, out_hbm, hist_v, buf0, buf1,
                    pref_v, sem0, sem1):
        wid = lax.axis_index("s") * NUM_CORES + lax.axis_index("c")
        base = wid * SHARD
        ones = jnp.ones((LANES,), jnp.int32)
        zeros = jnp.zeros((LANES,), jnp.int32)
        bufs = (buf0, buf1)
        sems = (sem0, sem1)

        pltpu.sync_copy(prefix_hbm, pref_v)
        pvec = pref_v[...]

        copies = [None] * NCHUNK
        copies[0] = pltpu.async_copy(
            logits_hbm.at[pl.ds(base, CHUNK)], buf0, sem0)

        @plsc.parallel_loop(0, BINS // LANES, 1, unroll=UNROLL)
        def _(j):
            hist_v[pl.ds(j * LANES, LANES)] = zeros

        for c in range(NCHUNK):
            if c + 1 < NCHUNK:
                copies[c + 1] = pltpu.async_copy(
                    logits_hbm.at[pl.ds(base + (c + 1) * CHUNK, CHUNK)],
                    bufs[(c + 1) % 2], sems[(c + 1) % 2])
            copies[c].wait()
            buf = bufs[c % 2]

            @plsc.parallel_loop(0, CHUNK // LANES, 1, unroll=UNROLL)
            def _(i):
                x = buf[pl.ds(i * LANES, LANES)]
                b = lax.bitcast_convert_type(x, jnp.int32)
                if not prefix_pass:
                    bins = lax.shift_right_logical(b, 16)
                    plsc.addupdate_scatter(hist_v, [bins], ones)
                else:
                    match = lax.shift_right_logical(b, 16) == pvec
                    bins = b & (BINS - 1)
                    plsc.addupdate_scatter(hist_v, [bins], ones, mask=match)

        pltpu.sync_copy(hist_v, out_hbm.at[wid])

    return hist_kernel


_hist_pass1 = _make_hist_kernel(prefix_pass=False)
_hist_pass2 = _make_hist_kernel(prefix_pass=True)


def _pick_bucket(h_key, rank):
    cum = jnp.cumsum(h_key)
    b = jnp.argmax(cum >= rank + 1).astype(jnp.int32)
    new_rank = rank - (cum[b] - h_key[b])
    return b, new_rank


def _sigmoid_body(x_ref, t_ref, o_ref):
    t = t_ref[0]
    z = (t - x_ref[...]) * jnp.float32(1.0 / TEMPERATURE)
    o_ref[...] = 1.0 / (1.0 + jnp.exp(z))


def kernel(logits):
    n = logits.shape[-1]
    rank = jnp.int32(n - K_TOP - 1)
    half = BINS // 2

    zeros16 = jnp.zeros((LANES,), jnp.int32)
    h1 = jnp.sum(_hist_pass1(logits, zeros16), axis=0)
    h1_key = jnp.concatenate([h1[half:][::-1], h1[:half]])
    b1, rank = _pick_bucket(h1_key, rank)
    neg = b1 < half
    raw_hi = jnp.where(neg, (BINS - 1) - b1, b1 - half)

    pref = jnp.full((LANES,), raw_hi, jnp.int32)
    h2 = jnp.sum(_hist_pass2(logits, pref), axis=0)
    h2_key = jnp.where(neg, h2[::-1], h2)
    b2, rank = _pick_bucket(h2_key, rank)

    key = (b1 << 16) | b2
    vbits = jnp.where(key < 0, key ^ jnp.int32(MIN_I32), ~key)
    kth_value = lax.bitcast_convert_type(vbits, jnp.float32)

    rows = 4096
    cols = n // rows
    block_rows = 512
    x2 = logits.reshape(rows, cols)
    t1 = kth_value.reshape(1)
    out = pl.pallas_call(
        _sigmoid_body,
        grid=(rows // block_rows,),
        in_specs=[
            pl.BlockSpec((block_rows, cols), lambda i: (i, 0)),
            pl.BlockSpec(memory_space=pltpu.SMEM),
        ],
        out_specs=pl.BlockSpec((block_rows, cols), lambda i: (i, 0)),
        out_shape=jax.ShapeDtypeStruct((rows, cols), jnp.float32),
    )(x2, t1)
    return out.reshape(n)
